# Initial kernel scaffold; baseline (speedup 1.0000x reference)
import jax
import jax.numpy as jnp
from jax.experimental import pallas as pl


def kernel(seq_len, table):
    raise NotImplementedError("write your pallas kernel here")


# speedup vs baseline: 110.0536x; 110.0536x over previous
import functools

import jax
import jax.numpy as jnp
from jax import lax
from jax.experimental import pallas as pl
from jax.experimental.pallas import tpu as pltpu
from jax.experimental.pallas import tpu_sc as plsc

NUM_HEADS = 16
MAX_REL = 128
SEQ = 2048
VOCAB = 2 * MAX_REL + 1
NUM_WORKERS = 32
ROWS_PER_W = NUM_HEADS * SEQ // NUM_WORKERS
BANK_W = 4096
BAND_START = 1904
BAND_W = 288
BAND_STRIDE = 384
FILL_W = BANK_W - BAND_START - BAND_W
FILL_STRIDE = 1920
STG_T0 = 8 * BAND_STRIDE
STG_T256 = STG_T0 + FILL_STRIDE
STG_N = STG_T256 + FILL_STRIDE
CHUNK = 8
NCHUNKS = ROWS_PER_W // CHUNK


def _build_kernel():
    mesh = plsc.VectorSubcoreMesh(core_axis_name="c", subcore_axis_name="s")

    @functools.partial(
        pl.kernel,
        mesh=mesh,
        out_type=jax.ShapeDtypeStruct((NUM_HEADS * SEQ * SEQ,), jnp.float32),
        scratch_types=[
            pltpu.VMEM((8 * BANK_W,), jnp.float32),
            pltpu.SemaphoreType.DMA,
            pltpu.SemaphoreType.DMA,
        ],
    )
    def _k(stg_hbm, out_hbm, banks_v, bsem, sem):
        wid = lax.axis_index("s") * 2 + lax.axis_index("c")

        bdescs = []
        for r in range(8):
            base = r * BANK_W
            bdescs.append(pltpu.async_copy(
                stg_hbm.at[pl.ds(STG_T0, FILL_W)],
                banks_v.at[pl.ds(base, FILL_W)], bsem))
            bdescs.append(pltpu.async_copy(
                stg_hbm.at[pl.ds(r * BAND_STRIDE, BAND_W)],
                banks_v.at[pl.ds(base + BAND_START, BAND_W)], bsem))
            bdescs.append(pltpu.async_copy(
                stg_hbm.at[pl.ds(STG_T256, FILL_W)],
                banks_v.at[pl.ds(base + BAND_START + BAND_W, FILL_W)], bsem))
        for d in bdescs:
            d.wait()

        base_row = wid * ROWS_PER_W

        def fire(c):
            descs = []
            for b in range(CHUNK):
                row = base_row + c * CHUNK + b
                i = row & (SEQ - 1)
                s = (SEQ - 1) - i
                r = s & 7
                q = pl.multiple_of(r * BANK_W + (s - r), 8)
                dst = pl.multiple_of(row * SEQ, 128)
                descs.append(
                    pltpu.async_copy(
                        banks_v.at[pl.ds(q, SEQ)],
                        out_hbm.at[pl.ds(dst, SEQ)], sem
                    )
                )
            return descs

        first = fire(0)

        def body(c, carry):
            for d in fire(c + 1):
                d.wait()
            return carry

        lax.fori_loop(0, NCHUNKS - 1, body, 0)
        for d in first:
            d.wait()

    return _k


_K = _build_kernel()


def kernel(seq_len, table):
    del seq_len
    t = table.reshape(VOCAB)
    parts = [jnp.pad(t, (15 - r, BAND_STRIDE - VOCAB - (15 - r)), mode="edge")
             for r in range(8)]
    parts.append(jnp.broadcast_to(t[0], (FILL_STRIDE,)))
    parts.append(jnp.broadcast_to(t[VOCAB - 1], (FILL_STRIDE,)))
    staged = jnp.concatenate(parts)
    out = _K(staged)
    return out.reshape(NUM_HEADS, SEQ, SEQ)


# speedup vs baseline: 110.5202x; 1.0042x over previous
import functools

import jax
import jax.numpy as jnp
from jax import lax
from jax.experimental import pallas as pl
from jax.experimental.pallas import tpu as pltpu
from jax.experimental.pallas import tpu_sc as plsc

NUM_HEADS = 16
MAX_REL = 128
SEQ = 2048
VOCAB = 2 * MAX_REL + 1
NUM_WORKERS = 32
ROWS_PER_W = NUM_HEADS * SEQ // NUM_WORKERS
BANK_W = 4096
BAND_START = 1904
BAND_W = 288
BAND_STRIDE = 384
FILL_W = BANK_W - BAND_START - BAND_W
FILL_STRIDE = 1920
STG_T0 = 8 * BAND_STRIDE
STG_T256 = STG_T0 + FILL_STRIDE
STG_N = STG_T256 + FILL_STRIDE
CHUNK = 8
NCHUNKS = ROWS_PER_W // CHUNK


def _build_kernel():
    mesh = plsc.VectorSubcoreMesh(core_axis_name="c", subcore_axis_name="s")

    @functools.partial(
        pl.kernel,
        mesh=mesh,
        out_type=jax.ShapeDtypeStruct((NUM_HEADS, SEQ, SEQ), jnp.float32),
        scratch_types=[
            pltpu.VMEM((8 * BANK_W,), jnp.float32),
            pltpu.SemaphoreType.DMA,
            pltpu.SemaphoreType.DMA,
        ],
        compiler_params=pltpu.CompilerParams(use_tc_tiling_on_sc=False),
    )
    def _k(stg_hbm, out_hbm, banks_v, bsem, sem):
        wid = lax.axis_index("s") * 2 + lax.axis_index("c")

        bdescs = []
        for r in range(8):
            base = r * BANK_W
            bdescs.append(pltpu.async_copy(
                stg_hbm.at[pl.ds(STG_T0, FILL_W)],
                banks_v.at[pl.ds(base, FILL_W)], bsem))
            bdescs.append(pltpu.async_copy(
                stg_hbm.at[pl.ds(r * BAND_STRIDE, BAND_W)],
                banks_v.at[pl.ds(base + BAND_START, BAND_W)], bsem))
            bdescs.append(pltpu.async_copy(
                stg_hbm.at[pl.ds(STG_T256, FILL_W)],
                banks_v.at[pl.ds(base + BAND_START + BAND_W, FILL_W)], bsem))
        for d in bdescs:
            d.wait()

        base_row = wid * ROWS_PER_W

        def fire(c):
            descs = []
            for b in range(CHUNK):
                row = base_row + c * CHUNK + b
                h = row >> 11
                i = row & (SEQ - 1)
                s = (SEQ - 1) - i
                r = s & 7
                q = pl.multiple_of(r * BANK_W + (s - r), 8)
                descs.append(
                    pltpu.async_copy(
                        banks_v.at[pl.ds(q, SEQ)],
                        out_hbm.at[h, i], sem
                    )
                )
            return descs

        first = fire(0)

        def body(c, carry):
            for d in fire(c + 1):
                d.wait()
            return carry

        lax.fori_loop(0, NCHUNKS - 1, body, 0)
        for d in first:
            d.wait()

    return _k


_K = _build_kernel()


def kernel(seq_len, table):
    del seq_len
    t = table.reshape(VOCAB)
    parts = [jnp.pad(t, (15 - r, BAND_STRIDE - VOCAB - (15 - r)), mode="edge")
             for r in range(8)]
    parts.append(jnp.broadcast_to(t[0], (FILL_STRIDE,)))
    parts.append(jnp.broadcast_to(t[VOCAB - 1], (FILL_STRIDE,)))
    staged = jnp.concatenate(parts)
    return _K(staged)


# speedup vs baseline: 352.7276x; 3.1915x over previous
import functools

import jax
import jax.numpy as jnp
from jax import lax
from jax.experimental import pallas as pl
from jax.experimental.pallas import tpu as pltpu
from jax.experimental.pallas import tpu_sc as plsc

NUM_HEADS = 16
MAX_REL = 128
SEQ = 2048
VOCAB = 2 * MAX_REL + 1
NUM_WORKERS = 32
GROUPS = NUM_HEADS * SEQ // 8
GROUPS_PER_W = GROUPS // NUM_WORKERS
BANK_W = 4096
BAND_START = 1904
BAND_W = 288
BAND_STRIDE = 384
FILL_W = BANK_W - BAND_START - BAND_W
FILL_STRIDE = 1920
STG_T0 = 8 * BAND_STRIDE
STG_T256 = STG_T0 + FILL_STRIDE
STG_N = STG_T256 + FILL_STRIDE


def _build_kernel():
    mesh = plsc.VectorSubcoreMesh(core_axis_name="c", subcore_axis_name="s")

    @functools.partial(
        pl.kernel,
        mesh=mesh,
        out_type=jax.ShapeDtypeStruct((NUM_HEADS, SEQ // 8, 8, SEQ),
                                      jnp.float32),
        scratch_types=[
            pltpu.VMEM((8 * BANK_W,), jnp.float32),
            pltpu.VMEM((2, 8, SEQ), jnp.float32),
            pltpu.SemaphoreType.DMA,
            pltpu.SemaphoreType.DMA,
        ],
    )
    def _k(stg_hbm, out_hbm, banks_v, grp_v, bsem, sem):
        wid = lax.axis_index("s") * 2 + lax.axis_index("c")

        bdescs = []
        for r in range(8):
            base = r * BANK_W
            bdescs.append(pltpu.async_copy(
                stg_hbm.at[pl.ds(STG_T0, FILL_W)],
                banks_v.at[pl.ds(base, FILL_W)], bsem))
            bdescs.append(pltpu.async_copy(
                stg_hbm.at[pl.ds(r * BAND_STRIDE, BAND_W)],
                banks_v.at[pl.ds(base + BAND_START, BAND_W)], bsem))
            bdescs.append(pltpu.async_copy(
                stg_hbm.at[pl.ds(STG_T256, FILL_W)],
                banks_v.at[pl.ds(base + BAND_START + BAND_W, FILL_W)], bsem))
        for d in bdescs:
            d.wait()

        g0 = wid * GROUPS_PER_W

        def fill(g, buf):
            i0 = (g << 3) & (SEQ - 1)
            s0 = (SEQ - 1) - i0
            q0 = pl.multiple_of(s0 - 7, 8)

            @plsc.parallel_loop(0, SEQ, step=32, unroll=2)
            def body(col):
                vals = []
                for k in range(8):
                    for u in range(2):
                        src = pl.multiple_of(
                            (7 - k) * BANK_W + q0 + col + u * 16, 8)
                        vals.append(banks_v[pl.ds(src, 16)])
                n = 0
                for k in range(8):
                    for u in range(2):
                        grp_v[buf, k, pl.ds(col + u * 16, 16)] = vals[n]
                        n += 1

        def fire(g, buf):
            h = g >> 8
            a = g & (SEQ // 8 - 1)
            return pltpu.async_copy(grp_v.at[buf], out_hbm.at[h, a], sem)

        fill(g0, 0)
        d_prev = fire(g0, 0)

        def body(n, carry):
            g = g0 + n + 1
            buf = (n + 1) & 1
            fill(g, buf)
            d = fire(g, buf)
            pltpu.make_async_copy(
                grp_v.at[0], out_hbm.at[0, 0], sem).wait()
            return carry

        lax.fori_loop(0, GROUPS_PER_W - 1, body, 0)
        pltpu.make_async_copy(grp_v.at[0], out_hbm.at[0, 0], sem).wait()
        del d_prev

    return _k


_K = _build_kernel()


def kernel(seq_len, table):
    del seq_len
    t = table.reshape(VOCAB)
    parts = [jnp.pad(t, (15 - r, BAND_STRIDE - VOCAB - (15 - r)), mode="edge")
             for r in range(8)]
    parts.append(jnp.broadcast_to(t[0], (FILL_STRIDE,)))
    parts.append(jnp.broadcast_to(t[VOCAB - 1], (FILL_STRIDE,)))
    staged = jnp.concatenate(parts)
    out = _K(staged)
    return out.reshape(NUM_HEADS, SEQ, SEQ)


# speedup vs baseline: 385.0245x; 1.0916x over previous
import functools

import jax
import jax.numpy as jnp
import numpy as np
from jax import lax
from jax.experimental import pallas as pl
from jax.experimental.pallas import tpu as pltpu
from jax.experimental.pallas import tpu_sc as plsc

NUM_HEADS = 16
MAX_REL = 128
SEQ = 2048
VOCAB = 2 * MAX_REL + 1
NUM_WORKERS = 32
GROUPS = NUM_HEADS * SEQ // 8
GROUPS_PER_W = GROUPS // NUM_WORKERS
BANK_W = 4096
BAND_START = 1904
BAND_W = 288
BAND_STRIDE = 384
FILL_W = BANK_W - BAND_START - BAND_W
FILL_STRIDE = 1920
STG_T0 = 8 * BAND_STRIDE
STG_T256 = STG_T0 + FILL_STRIDE
STG_N = STG_T256 + FILL_STRIDE


def _build_kernel():
    mesh = plsc.VectorSubcoreMesh(core_axis_name="c", subcore_axis_name="s")

    @functools.partial(
        pl.kernel,
        mesh=mesh,
        out_type=jax.ShapeDtypeStruct((NUM_HEADS, SEQ // 8, 8, SEQ),
                                      jnp.float32),
        scratch_types=[
            pltpu.VMEM((8 * BANK_W,), jnp.float32),
            pltpu.VMEM((2, 8, SEQ), jnp.float32),
            pltpu.SemaphoreType.DMA,
            pltpu.SemaphoreType.DMA,
        ],
    )
    def _k(stg_hbm, out_hbm, banks_v, grp_v, bsem, sem):
        wid = lax.axis_index("s") * 2 + lax.axis_index("c")

        bdescs = []
        for r in range(8):
            base = r * BANK_W
            bdescs.append(pltpu.async_copy(
                stg_hbm.at[pl.ds(STG_T0, FILL_W)],
                banks_v.at[pl.ds(base, FILL_W)], bsem))
            bdescs.append(pltpu.async_copy(
                stg_hbm.at[pl.ds(r * BAND_STRIDE, BAND_W)],
                banks_v.at[pl.ds(base + BAND_START, BAND_W)], bsem))
            bdescs.append(pltpu.async_copy(
                stg_hbm.at[pl.ds(STG_T256, FILL_W)],
                banks_v.at[pl.ds(base + BAND_START + BAND_W, FILL_W)], bsem))
        for d in bdescs:
            d.wait()

        g0 = wid * GROUPS_PER_W

        def fill(g, buf):
            i0 = (g << 3) & (SEQ - 1)
            s0 = (SEQ - 1) - i0
            q0 = pl.multiple_of(s0 - 7, 8)

            @plsc.parallel_loop(0, SEQ, step=32, unroll=2)
            def body(col):
                vals = []
                for k in range(8):
                    for u in range(2):
                        src = pl.multiple_of(
                            (7 - k) * BANK_W + q0 + col + u * 16, 8)
                        vals.append(banks_v[pl.ds(src, 16)])
                n = 0
                for k in range(8):
                    for u in range(2):
                        grp_v[buf, k, pl.ds(col + u * 16, 16)] = vals[n]
                        n += 1

        def fire(g, buf):
            h = g >> 8
            a = g & (SEQ // 8 - 1)
            return pltpu.async_copy(grp_v.at[buf], out_hbm.at[h, a], sem)

        fill(g0, 0)
        d_prev = fire(g0, 0)

        def body(n, carry):
            g = g0 + n + 1
            buf = (n + 1) & 1
            fill(g, buf)
            d = fire(g, buf)
            pltpu.make_async_copy(
                grp_v.at[0], out_hbm.at[0, 0], sem).wait()
            return carry

        lax.fori_loop(0, GROUPS_PER_W - 1, body, 0)
        pltpu.make_async_copy(grp_v.at[0], out_hbm.at[0, 0], sem).wait()
        del d_prev

    return _k


_K = _build_kernel()


def _staging_matrix():
    idx = np.empty((STG_N,), dtype=np.int64)
    for r in range(8):
        base = r * BAND_STRIDE
        idx[base:base + BAND_STRIDE] = np.clip(
            np.arange(BAND_STRIDE) - (15 - r), 0, VOCAB - 1)
    idx[STG_T0:STG_T0 + FILL_STRIDE] = 0
    idx[STG_T256:STG_T256 + FILL_STRIDE] = VOCAB - 1
    return np.eye(VOCAB, dtype=np.float32)[idx]


_SEL = _staging_matrix()


def kernel(seq_len, table):
    del seq_len
    staged = (_SEL @ table).reshape(STG_N)
    out = _K(staged)
    return out.reshape(NUM_HEADS, SEQ, SEQ)
